# baseline (speedup 1.0000x reference)
import jax
import jax.numpy as jnp
from jax import lax
from jax.experimental import pallas as pl
from jax.experimental.pallas import tpu as pltpu

_ROWS, _COLS = 128, 32768
_RB = 64
_CB = _COLS // 2
_NRB = _ROWS // _RB


def _body(x_ref, o_ref, cnt_ref, piv_ref):
    s = pl.program_id(0)
    slot = jnp.minimum(s // 2, _NRB - 1)

    @pl.when(s < 4)
    def _():
        x = x_ref[...]
        first = s % 2 == 0

        @pl.when(first)
        def _():
            pivot = x[:, 0:1]
            gt = (x > pivot).astype(jnp.int32)
            part = jnp.sum(gt, axis=1, keepdims=True)
            cnt_ref[slot] = jnp.broadcast_to(part, (_RB, 128))
            piv_ref[slot] = jnp.broadcast_to(pivot, (_RB, 128))

        @pl.when(jnp.logical_not(first))
        def _():
            pivot = piv_ref[slot][:, 0:1]
            gt = (x > pivot).astype(jnp.int32)
            part = jnp.sum(gt, axis=1, keepdims=True)
            cnt_ref[slot] = cnt_ref[slot] + jnp.broadcast_to(part, (_RB, 128))

    @pl.when(s >= 1)
    def _():
        wslot = jnp.minimum((s - 1) // 2, _NRB - 1)
        off = jnp.where(s % 2 == 1, 0, _CB)
        cnt = cnt_ref[wslot][:, 0:1]
        iota = lax.broadcasted_iota(jnp.int32, (_RB, _CB), 1) + off
        o_ref[...] = (iota == cnt).astype(jnp.float32)


def _in_map(s):
    rb = jnp.minimum(s // 2, _NRB - 1)
    cb = jnp.where(s >= 4, 1, s % 2)
    return rb, cb


def _out_map(s):
    rb = jnp.where(s < 3, 0, 1)
    cb = jnp.where(s < 2, 0, jnp.where(s == 2, 1, jnp.where(s == 3, 0, 1)))
    return rb, cb


def kernel(scores):
    return pl.pallas_call(
        _body,
        grid=(2 * _NRB + 1,),
        in_specs=[pl.BlockSpec((_RB, _CB), _in_map)],
        out_specs=pl.BlockSpec((_RB, _CB), _out_map),
        out_shape=jax.ShapeDtypeStruct((_ROWS, _COLS), jnp.float32),
        scratch_shapes=[
            pltpu.VMEM((_NRB, _RB, 128), jnp.int32),
            pltpu.VMEM((_NRB, _RB, 128), jnp.float32),
        ],
    )(scores)


# speedup vs baseline: 1.2334x; 1.2334x over previous
import jax
import jax.numpy as jnp
from jax.experimental import pallas as pl

_ROWS, _COLS = 128, 32768
_BLOCK_ROWS = 64


def _onehot_rank_body(x_ref, o_ref):
    x = x_ref[...]
    pivot = x[:, 0:1]
    gt = (x > pivot).astype(jnp.int32)
    cnt = jnp.sum(gt, axis=1, keepdims=True)
    iota = jax.lax.broadcasted_iota(jnp.int32, x.shape, 1)
    o_ref[...] = (iota == cnt).astype(jnp.float32)


def kernel(scores):
    return pl.pallas_call(
        _onehot_rank_body,
        grid=(_ROWS // _BLOCK_ROWS,),
        in_specs=[pl.BlockSpec((_BLOCK_ROWS, _COLS), lambda i: (i, 0))],
        out_specs=pl.BlockSpec((_BLOCK_ROWS, _COLS), lambda i: (i, 0)),
        out_shape=jax.ShapeDtypeStruct((_ROWS, _COLS), jnp.float32),
    )(scores)
